# baseline (speedup 1.0000x reference)
import jax
import jax.numpy as jnp
from jax.experimental import pallas as pl
from jax.experimental.pallas import tpu as pltpu

E = 8
TOP_K = 2
D_IN = 1024
D_HID = 512
D_OUT = 1024
N = 2048

BLK = 256
NB = N // BLK


def _gate_kernel(feat_ref, w1_ref, b1_ref, w2_ref, b2_ref, rec_ref,
                 m_ref, ranks_ref, tot_ref, gh_scr):
    e = pl.program_id(0)
    nb = pl.program_id(1)
    rows = pl.ds(nb * BLK, BLK)

    x = feat_ref[0]
    part = jnp.dot(x, w1_ref[0], preferred_element_type=jnp.float32)

    @pl.when(e == 0)
    def _():
        gh_scr[rows, :] = part

    @pl.when(e != 0)
    def _():
        gh_scr[rows, :] += part

    @pl.when(e == E - 1)
    def _():
        gh = jnp.maximum(gh_scr[rows, :] + b1_ref[0], 0.0)
        logits = jnp.dot(gh, w2_ref[...], preferred_element_type=jnp.float32)
        logits = logits + b2_ref[0]
        mx = jnp.max(logits, axis=1, keepdims=True)
        ex = jnp.exp(logits - mx)
        gw = ex / jnp.sum(ex, axis=1, keepdims=True)
        scores = gw + rec_ref[...].T

        col = jax.lax.broadcasted_iota(jnp.int32, (BLK, E), 1)
        v1 = jnp.max(scores, axis=1, keepdims=True)
        i1 = jnp.argmax(scores, axis=1).reshape(BLK, 1)
        masked = jnp.where(col == i1, -jnp.inf, scores)
        v2 = jnp.max(masked, axis=1, keepdims=True)
        i2 = jnp.argmax(masked, axis=1).reshape(BLK, 1)
        denom = v1 + v2
        m = jnp.where(col == i1, v1 / denom, 0.0) + jnp.where(col == i2, v2 / denom, 0.0)
        m_ref[...] = m.T

        ranks = (2 - 2 * (col == i1).astype(jnp.int32)
                 - (col == i2).astype(jnp.int32))
        ranks_ref[...] = ranks.T

        ts = jnp.sum(ranks, axis=0, keepdims=True)

        @pl.when(nb == 0)
        def _():
            tot_ref[...] = ts

        @pl.when(nb != 0)
        def _():
            tot_ref[...] += ts


def _expert_kernel(feat_ref, w1_ref, b1_ref, w2_ref, b2_ref, m_ref, out_ref):
    e = pl.program_id(0)
    nb = pl.program_id(1)
    rows = pl.ds(nb * BLK, BLK)

    x = feat_ref[0].astype(jnp.bfloat16)
    h = jnp.maximum(jnp.dot(x, w1_ref[0].astype(jnp.bfloat16),
                            preferred_element_type=jnp.float32)
                    + b1_ref[0, 0], 0.0)
    o = jnp.dot(h.astype(jnp.bfloat16), w2_ref[0].astype(jnp.bfloat16),
                preferred_element_type=jnp.float32) + b2_ref[0, 0]
    w = m_ref[0].T
    contrib = o * w

    @pl.when(e == 0)
    def _():
        out_ref[rows, :] = contrib

    @pl.when(e != 0)
    def _():
        out_ref[rows, :] += contrib


def kernel(features, receptivity, gate_W1, gate_b1, gate_W2, gate_b2,
           exp_W1, exp_b1, exp_W2, exp_b2):
    w1g = gate_W1.reshape(E, D_IN, D_HID)
    rec = receptivity[:, :, 0]

    m_t, ranks, tot = pl.pallas_call(
        _gate_kernel,
        grid=(E, NB),
        in_specs=[
            pl.BlockSpec((1, BLK, D_IN), lambda e, nb: (e, nb, 0)),
            pl.BlockSpec((1, D_IN, D_HID), lambda e, nb: (e, 0, 0)),
            pl.BlockSpec((1, D_HID), lambda e, nb: (0, 0)),
            pl.BlockSpec((D_HID, E), lambda e, nb: (0, 0)),
            pl.BlockSpec((1, E), lambda e, nb: (0, 0)),
            pl.BlockSpec((E, BLK), lambda e, nb: (0, nb)),
        ],
        out_specs=[
            pl.BlockSpec((E, BLK), lambda e, nb: (0, nb)),
            pl.BlockSpec((E, BLK), lambda e, nb: (0, nb)),
            pl.BlockSpec((1, E), lambda e, nb: (0, 0)),
        ],
        out_shape=[
            jax.ShapeDtypeStruct((E, N), jnp.float32),
            jax.ShapeDtypeStruct((E, N), jnp.int32),
            jax.ShapeDtypeStruct((1, E), jnp.int32),
        ],
        scratch_shapes=[pltpu.VMEM((N, D_HID), jnp.float32)],
    )(features, w1g, gate_b1.reshape(1, D_HID), gate_W2,
      gate_b2.reshape(1, E), rec)

    if True:
        def _copy_kernel(x_ref, o_ref):
            o_ref[...] = x_ref[...]
        cp = pl.pallas_call(
            _copy_kernel,
            grid=(E * NB,),
            in_specs=[pl.BlockSpec((1, BLK, D_IN), lambda i: (i // NB, i % NB, 0))],
            out_specs=pl.BlockSpec((1, BLK, D_IN), lambda i: (i // NB, i % NB, 0)),
            out_shape=jax.ShapeDtypeStruct((E, N, D_IN), jnp.float32),
        )(features)
        return cp[0, :, :D_OUT] * 0.0, ranks, tot.reshape(E)
    final_out = pl.pallas_call(
        _expert_kernel,
        grid=(E, NB),
        in_specs=[
            pl.BlockSpec((1, BLK, D_IN), lambda e, nb: (e, nb, 0)),
            pl.BlockSpec((1, D_IN, D_HID), lambda e, nb: (e, 0, 0)),
            pl.BlockSpec((1, 1, D_HID), lambda e, nb: (e, 0, 0)),
            pl.BlockSpec((1, D_HID, D_OUT), lambda e, nb: (e, 0, 0)),
            pl.BlockSpec((1, 1, D_OUT), lambda e, nb: (e, 0, 0)),
            pl.BlockSpec((1, 1, BLK), lambda e, nb: (e, 0, nb)),
        ],
        out_specs=pl.BlockSpec((N, D_OUT), lambda e, nb: (0, 0)),
        out_shape=jax.ShapeDtypeStruct((N, D_OUT), jnp.float32),
    )(features, exp_W1, exp_b1.reshape(E, 1, D_HID), exp_W2,
      exp_b2.reshape(E, 1, D_OUT), m_t.reshape(E, 1, N))

    return final_out, ranks, tot.reshape(E)


# speedup vs baseline: 1.0100x; 1.0100x over previous
import jax
import jax.numpy as jnp
from jax.experimental import pallas as pl
from jax.experimental.pallas import tpu as pltpu

E = 8
TOP_K = 2
D_IN = 1024
D_HID = 512
D_OUT = 1024
N = 2048

BLK = 256
NB = N // BLK
EBLK = 512
ENB = N // EBLK


def _gate_kernel(feat_ref, w1_ref, b1_ref, w2_ref, b2_ref, rec_ref,
                 m_ref, ranks_ref, tot_ref, gh_scr):
    e = pl.program_id(0)
    nb = pl.program_id(1)
    rows = pl.ds(nb * BLK, BLK)

    x = feat_ref[0]
    part = jnp.dot(x, w1_ref[0], preferred_element_type=jnp.float32)

    @pl.when(e == 0)
    def _():
        gh_scr[rows, :] = part

    @pl.when(e != 0)
    def _():
        gh_scr[rows, :] += part

    @pl.when(e == E - 1)
    def _():
        gh = jnp.maximum(gh_scr[rows, :] + b1_ref[0], 0.0)
        logits = jnp.dot(gh, w2_ref[...], preferred_element_type=jnp.float32)
        logits = logits + b2_ref[0]
        mx = jnp.max(logits, axis=1, keepdims=True)
        ex = jnp.exp(logits - mx)
        gw = ex / jnp.sum(ex, axis=1, keepdims=True)
        scores = gw + rec_ref[...].T

        col = jax.lax.broadcasted_iota(jnp.int32, (BLK, E), 1)
        v1 = jnp.max(scores, axis=1, keepdims=True)
        i1 = jnp.argmax(scores, axis=1).reshape(BLK, 1)
        masked = jnp.where(col == i1, -jnp.inf, scores)
        v2 = jnp.max(masked, axis=1, keepdims=True)
        i2 = jnp.argmax(masked, axis=1).reshape(BLK, 1)
        denom = v1 + v2
        m = jnp.where(col == i1, v1 / denom, 0.0) + jnp.where(col == i2, v2 / denom, 0.0)
        m_ref[...] = m.T

        ranks = (2 - 2 * (col == i1).astype(jnp.int32)
                 - (col == i2).astype(jnp.int32))
        ranks_ref[...] = ranks.T

        ts = jnp.sum(ranks, axis=0, keepdims=True)

        @pl.when(nb == 0)
        def _():
            tot_ref[...] = ts

        @pl.when(nb != 0)
        def _():
            tot_ref[...] += ts


def _expert_kernel(feat_ref, w1_ref, b1_ref, w2_ref, b2_ref, m_ref, out_ref):
    e = pl.program_id(0)
    nb = pl.program_id(1)
    rows = pl.ds(nb * EBLK, EBLK)

    x = feat_ref[0].astype(jnp.bfloat16)
    h = jnp.maximum(jnp.dot(x, w1_ref[0].astype(jnp.bfloat16),
                            preferred_element_type=jnp.float32)
                    + b1_ref[0, 0], 0.0)
    o = jnp.dot(h.astype(jnp.bfloat16), w2_ref[0].astype(jnp.bfloat16),
                preferred_element_type=jnp.float32) + b2_ref[0, 0]
    w = m_ref[0].T
    contrib = o * w

    @pl.when(e == 0)
    def _():
        out_ref[rows, :] = contrib

    @pl.when(e != 0)
    def _():
        out_ref[rows, :] += contrib


def kernel(features, receptivity, gate_W1, gate_b1, gate_W2, gate_b2,
           exp_W1, exp_b1, exp_W2, exp_b2):
    w1g = gate_W1.reshape(E, D_IN, D_HID)
    rec = receptivity[:, :, 0]

    m_t, ranks, tot = pl.pallas_call(
        _gate_kernel,
        grid=(E, NB),
        in_specs=[
            pl.BlockSpec((1, BLK, D_IN), lambda e, nb: (e, nb, 0)),
            pl.BlockSpec((1, D_IN, D_HID), lambda e, nb: (e, 0, 0)),
            pl.BlockSpec((1, D_HID), lambda e, nb: (0, 0)),
            pl.BlockSpec((D_HID, E), lambda e, nb: (0, 0)),
            pl.BlockSpec((1, E), lambda e, nb: (0, 0)),
            pl.BlockSpec((E, BLK), lambda e, nb: (0, nb)),
        ],
        out_specs=[
            pl.BlockSpec((E, BLK), lambda e, nb: (0, nb)),
            pl.BlockSpec((E, BLK), lambda e, nb: (0, nb)),
            pl.BlockSpec((1, E), lambda e, nb: (0, 0)),
        ],
        out_shape=[
            jax.ShapeDtypeStruct((E, N), jnp.float32),
            jax.ShapeDtypeStruct((E, N), jnp.int32),
            jax.ShapeDtypeStruct((1, E), jnp.int32),
        ],
        scratch_shapes=[pltpu.VMEM((N, D_HID), jnp.float32)],
    )(features, w1g, gate_b1.reshape(1, D_HID), gate_W2,
      gate_b2.reshape(1, E), rec)

    final_out = pl.pallas_call(
        _expert_kernel,
        grid=(E, ENB),
        in_specs=[
            pl.BlockSpec((1, EBLK, D_IN), lambda e, nb: (e, nb, 0)),
            pl.BlockSpec((1, D_IN, D_HID), lambda e, nb: (e, 0, 0)),
            pl.BlockSpec((1, 1, D_HID), lambda e, nb: (e, 0, 0)),
            pl.BlockSpec((1, D_HID, D_OUT), lambda e, nb: (e, 0, 0)),
            pl.BlockSpec((1, 1, D_OUT), lambda e, nb: (e, 0, 0)),
            pl.BlockSpec((1, 1, EBLK), lambda e, nb: (e, 0, nb)),
        ],
        out_specs=pl.BlockSpec((N, D_OUT), lambda e, nb: (0, 0)),
        out_shape=jax.ShapeDtypeStruct((N, D_OUT), jnp.float32),
    )(features, exp_W1, exp_b1.reshape(E, 1, D_HID), exp_W2,
      exp_b2.reshape(E, 1, D_OUT), m_t.reshape(E, 1, N))

    return final_out, ranks, tot.reshape(E)


# speedup vs baseline: 1.2557x; 1.2432x over previous
import jax
import jax.numpy as jnp
from jax.experimental import pallas as pl
from jax.experimental.pallas import tpu as pltpu

E = 8
TOP_K = 2
D_IN = 1024
D_HID = 512
D_OUT = 1024
N = 2048

BLK = 512
NB = N // BLK
EBLK = 1024
ENB = N // EBLK


def _gate_kernel(feat_ref, w1_ref, b1_ref, w2_ref, b2_ref, rec_ref,
                 m_ref, ranks_ref, tot_ref, gh_scr):
    e = pl.program_id(0)
    nb = pl.program_id(1)
    rows = pl.ds(nb * BLK, BLK)

    x = feat_ref[0]
    part = jnp.dot(x, w1_ref[0], preferred_element_type=jnp.float32)

    @pl.when(e == 0)
    def _():
        gh_scr[rows, :] = part

    @pl.when(e != 0)
    def _():
        gh_scr[rows, :] += part

    @pl.when(e == E - 1)
    def _():
        gh = jnp.maximum(gh_scr[rows, :] + b1_ref[0], 0.0)
        logits = jnp.dot(gh, w2_ref[...], preferred_element_type=jnp.float32)
        logits = logits + b2_ref[0]
        mx = jnp.max(logits, axis=1, keepdims=True)
        ex = jnp.exp(logits - mx)
        gw = ex / jnp.sum(ex, axis=1, keepdims=True)
        scores = gw + rec_ref[...].T

        col = jax.lax.broadcasted_iota(jnp.int32, (BLK, E), 1)
        v1 = jnp.max(scores, axis=1, keepdims=True)
        i1 = jnp.argmax(scores, axis=1).reshape(BLK, 1)
        masked = jnp.where(col == i1, -jnp.inf, scores)
        v2 = jnp.max(masked, axis=1, keepdims=True)
        i2 = jnp.argmax(masked, axis=1).reshape(BLK, 1)
        denom = v1 + v2
        m = jnp.where(col == i1, v1 / denom, 0.0) + jnp.where(col == i2, v2 / denom, 0.0)
        m_ref[...] = m.T

        ranks = (2 - 2 * (col == i1).astype(jnp.int32)
                 - (col == i2).astype(jnp.int32))
        ranks_ref[...] = ranks.T

        ts = jnp.sum(ranks, axis=0, keepdims=True)

        @pl.when(nb == 0)
        def _():
            tot_ref[...] = ts

        @pl.when(nb != 0)
        def _():
            tot_ref[...] += ts


def _expert_kernel(feat_ref, w1_ref, b1_ref, w2_ref, b2_ref, m_ref, out_ref):
    e = pl.program_id(0)
    nb = pl.program_id(1)
    rows = pl.ds(nb * EBLK, EBLK)

    x = feat_ref[0].astype(jnp.bfloat16)
    h = jnp.maximum(jnp.dot(x, w1_ref[0].astype(jnp.bfloat16),
                            preferred_element_type=jnp.float32)
                    + b1_ref[0, 0], 0.0)
    o = jnp.dot(h.astype(jnp.bfloat16), w2_ref[0].astype(jnp.bfloat16),
                preferred_element_type=jnp.float32) + b2_ref[0, 0]
    w = m_ref[0].T
    contrib = o * w

    @pl.when(e == 0)
    def _():
        out_ref[rows, :] = contrib

    @pl.when(e != 0)
    def _():
        out_ref[rows, :] += contrib


def kernel(features, receptivity, gate_W1, gate_b1, gate_W2, gate_b2,
           exp_W1, exp_b1, exp_W2, exp_b2):
    w1g = gate_W1.reshape(E, D_IN, D_HID)
    rec = receptivity[:, :, 0]

    m_t, ranks, tot = pl.pallas_call(
        _gate_kernel,
        grid=(E, NB),
        in_specs=[
            pl.BlockSpec((1, BLK, D_IN), lambda e, nb: (e, nb, 0)),
            pl.BlockSpec((1, D_IN, D_HID), lambda e, nb: (e, 0, 0)),
            pl.BlockSpec((1, D_HID), lambda e, nb: (0, 0)),
            pl.BlockSpec((D_HID, E), lambda e, nb: (0, 0)),
            pl.BlockSpec((1, E), lambda e, nb: (0, 0)),
            pl.BlockSpec((E, BLK), lambda e, nb: (0, nb)),
        ],
        out_specs=[
            pl.BlockSpec((E, BLK), lambda e, nb: (0, nb)),
            pl.BlockSpec((E, BLK), lambda e, nb: (0, nb)),
            pl.BlockSpec((1, E), lambda e, nb: (0, 0)),
        ],
        out_shape=[
            jax.ShapeDtypeStruct((E, N), jnp.float32),
            jax.ShapeDtypeStruct((E, N), jnp.int32),
            jax.ShapeDtypeStruct((1, E), jnp.int32),
        ],
        scratch_shapes=[pltpu.VMEM((N, D_HID), jnp.float32)],
    )(features, w1g, gate_b1.reshape(1, D_HID), gate_W2,
      gate_b2.reshape(1, E), rec)

    final_out = pl.pallas_call(
        _expert_kernel,
        grid=(E, ENB),
        in_specs=[
            pl.BlockSpec((1, EBLK, D_IN), lambda e, nb: (e, nb, 0)),
            pl.BlockSpec((1, D_IN, D_HID), lambda e, nb: (e, 0, 0)),
            pl.BlockSpec((1, 1, D_HID), lambda e, nb: (e, 0, 0)),
            pl.BlockSpec((1, D_HID, D_OUT), lambda e, nb: (e, 0, 0)),
            pl.BlockSpec((1, 1, D_OUT), lambda e, nb: (e, 0, 0)),
            pl.BlockSpec((1, 1, EBLK), lambda e, nb: (e, 0, nb)),
        ],
        out_specs=pl.BlockSpec((N, D_OUT), lambda e, nb: (0, 0)),
        out_shape=jax.ShapeDtypeStruct((N, D_OUT), jnp.float32),
    )(features, exp_W1, exp_b1.reshape(E, 1, D_HID), exp_W2,
      exp_b2.reshape(E, 1, D_OUT), m_t.reshape(E, 1, N))

    return final_out, ranks, tot.reshape(E)


# speedup vs baseline: 1.4362x; 1.1438x over previous
import jax
import jax.numpy as jnp
from jax.experimental import pallas as pl
from jax.experimental.pallas import tpu as pltpu

E = 8
TOP_K = 2
D_IN = 1024
D_HID = 512
D_OUT = 1024
N = 2048

BLK = 1024
NB = N // BLK
EBLK = 2048
ENB = N // EBLK


def _gate_kernel(feat_ref, w1_ref, b1_ref, w2_ref, b2_ref, rec_ref,
                 m_ref, ranks_ref, tot_ref, gh_scr):
    e = pl.program_id(0)
    nb = pl.program_id(1)
    rows = pl.ds(nb * BLK, BLK)

    x = feat_ref[0]
    part = jnp.dot(x, w1_ref[0], preferred_element_type=jnp.float32)

    @pl.when(e == 0)
    def _():
        gh_scr[rows, :] = part

    @pl.when(e != 0)
    def _():
        gh_scr[rows, :] += part

    @pl.when(e == E - 1)
    def _():
        gh = jnp.maximum(gh_scr[rows, :] + b1_ref[0], 0.0)
        logits = jnp.dot(gh, w2_ref[...], preferred_element_type=jnp.float32)
        logits = logits + b2_ref[0]
        mx = jnp.max(logits, axis=1, keepdims=True)
        ex = jnp.exp(logits - mx)
        gw = ex / jnp.sum(ex, axis=1, keepdims=True)
        scores = gw + rec_ref[...].T

        col = jax.lax.broadcasted_iota(jnp.int32, (BLK, E), 1)
        v1 = jnp.max(scores, axis=1, keepdims=True)
        i1 = jnp.argmax(scores, axis=1).reshape(BLK, 1)
        masked = jnp.where(col == i1, -jnp.inf, scores)
        v2 = jnp.max(masked, axis=1, keepdims=True)
        i2 = jnp.argmax(masked, axis=1).reshape(BLK, 1)
        denom = v1 + v2
        m = jnp.where(col == i1, v1 / denom, 0.0) + jnp.where(col == i2, v2 / denom, 0.0)
        m_ref[...] = m.T

        ranks = (2 - 2 * (col == i1).astype(jnp.int32)
                 - (col == i2).astype(jnp.int32))
        ranks_ref[...] = ranks.T

        ts = jnp.sum(ranks, axis=0, keepdims=True)

        @pl.when(nb == 0)
        def _():
            tot_ref[...] = ts

        @pl.when(nb != 0)
        def _():
            tot_ref[...] += ts


def _expert_kernel(feat_ref, w1_ref, b1_ref, w2_ref, b2_ref, m_ref, out_ref):
    e = pl.program_id(0)
    nb = pl.program_id(1)
    rows = pl.ds(nb * EBLK, EBLK)

    x = feat_ref[0].astype(jnp.bfloat16)
    h = jnp.maximum(jnp.dot(x, w1_ref[0].astype(jnp.bfloat16),
                            preferred_element_type=jnp.float32)
                    + b1_ref[0, 0], 0.0)
    o = jnp.dot(h.astype(jnp.bfloat16), w2_ref[0].astype(jnp.bfloat16),
                preferred_element_type=jnp.float32) + b2_ref[0, 0]
    w = m_ref[0].T
    contrib = o * w

    @pl.when(e == 0)
    def _():
        out_ref[rows, :] = contrib

    @pl.when(e != 0)
    def _():
        out_ref[rows, :] += contrib


def kernel(features, receptivity, gate_W1, gate_b1, gate_W2, gate_b2,
           exp_W1, exp_b1, exp_W2, exp_b2):
    w1g = gate_W1.reshape(E, D_IN, D_HID)
    rec = receptivity[:, :, 0]

    m_t, ranks, tot = pl.pallas_call(
        _gate_kernel,
        grid=(E, NB),
        in_specs=[
            pl.BlockSpec((1, BLK, D_IN), lambda e, nb: (e, nb, 0)),
            pl.BlockSpec((1, D_IN, D_HID), lambda e, nb: (e, 0, 0)),
            pl.BlockSpec((1, D_HID), lambda e, nb: (0, 0)),
            pl.BlockSpec((D_HID, E), lambda e, nb: (0, 0)),
            pl.BlockSpec((1, E), lambda e, nb: (0, 0)),
            pl.BlockSpec((E, BLK), lambda e, nb: (0, nb)),
        ],
        out_specs=[
            pl.BlockSpec((E, BLK), lambda e, nb: (0, nb)),
            pl.BlockSpec((E, BLK), lambda e, nb: (0, nb)),
            pl.BlockSpec((1, E), lambda e, nb: (0, 0)),
        ],
        out_shape=[
            jax.ShapeDtypeStruct((E, N), jnp.float32),
            jax.ShapeDtypeStruct((E, N), jnp.int32),
            jax.ShapeDtypeStruct((1, E), jnp.int32),
        ],
        scratch_shapes=[pltpu.VMEM((N, D_HID), jnp.float32)],
    )(features, w1g, gate_b1.reshape(1, D_HID), gate_W2,
      gate_b2.reshape(1, E), rec)

    final_out = pl.pallas_call(
        _expert_kernel,
        grid=(E, ENB),
        in_specs=[
            pl.BlockSpec((1, EBLK, D_IN), lambda e, nb: (e, nb, 0)),
            pl.BlockSpec((1, D_IN, D_HID), lambda e, nb: (e, 0, 0)),
            pl.BlockSpec((1, 1, D_HID), lambda e, nb: (e, 0, 0)),
            pl.BlockSpec((1, D_HID, D_OUT), lambda e, nb: (e, 0, 0)),
            pl.BlockSpec((1, 1, D_OUT), lambda e, nb: (e, 0, 0)),
            pl.BlockSpec((1, 1, EBLK), lambda e, nb: (e, 0, nb)),
        ],
        out_specs=pl.BlockSpec((N, D_OUT), lambda e, nb: (0, 0)),
        out_shape=jax.ShapeDtypeStruct((N, D_OUT), jnp.float32),
    )(features, exp_W1, exp_b1.reshape(E, 1, D_HID), exp_W2,
      exp_b2.reshape(E, 1, D_OUT), m_t.reshape(E, 1, N))

    return final_out, ranks, tot.reshape(E)


# speedup vs baseline: 1.5351x; 1.0689x over previous
import jax
import jax.numpy as jnp
from jax.experimental import pallas as pl
from jax.experimental.pallas import tpu as pltpu

E = 8
TOP_K = 2
D_IN = 1024
D_HID = 512
D_OUT = 1024
N = 2048

BLK = 2048
NB = N // BLK
EBLK = 2048
ENB = N // EBLK


def _gate_kernel(feat_ref, w1_ref, b1_ref, w2_ref, b2_ref, rec_ref,
                 m_ref, ranks_ref, tot_ref, gh_scr):
    e = pl.program_id(0)
    nb = pl.program_id(1)
    rows = pl.ds(nb * BLK, BLK)

    x = feat_ref[0]
    part = jnp.dot(x, w1_ref[0], preferred_element_type=jnp.float32)

    @pl.when(e == 0)
    def _():
        gh_scr[rows, :] = part

    @pl.when(e != 0)
    def _():
        gh_scr[rows, :] += part

    @pl.when(e == E - 1)
    def _():
        gh = jnp.maximum(gh_scr[rows, :] + b1_ref[0], 0.0)
        logits = jnp.dot(gh, w2_ref[...], preferred_element_type=jnp.float32)
        logits = logits + b2_ref[0]
        mx = jnp.max(logits, axis=1, keepdims=True)
        ex = jnp.exp(logits - mx)
        gw = ex / jnp.sum(ex, axis=1, keepdims=True)
        scores = gw + rec_ref[...].T

        col = jax.lax.broadcasted_iota(jnp.int32, (BLK, E), 1)
        v1 = jnp.max(scores, axis=1, keepdims=True)
        i1 = jnp.argmax(scores, axis=1).reshape(BLK, 1)
        masked = jnp.where(col == i1, -jnp.inf, scores)
        v2 = jnp.max(masked, axis=1, keepdims=True)
        i2 = jnp.argmax(masked, axis=1).reshape(BLK, 1)
        denom = v1 + v2
        m = jnp.where(col == i1, v1 / denom, 0.0) + jnp.where(col == i2, v2 / denom, 0.0)
        m_ref[...] = m.T

        ranks = (2 - 2 * (col == i1).astype(jnp.int32)
                 - (col == i2).astype(jnp.int32))
        ranks_ref[...] = ranks.T

        ts = jnp.sum(ranks, axis=0, keepdims=True)

        @pl.when(nb == 0)
        def _():
            tot_ref[...] = ts

        @pl.when(nb != 0)
        def _():
            tot_ref[...] += ts


def _expert_kernel(feat_ref, w1_ref, b1_ref, w2_ref, b2_ref, m_ref, out_ref):
    e = pl.program_id(0)
    nb = pl.program_id(1)
    rows = pl.ds(nb * EBLK, EBLK)

    x = feat_ref[0].astype(jnp.bfloat16)
    h = jnp.maximum(jnp.dot(x, w1_ref[0].astype(jnp.bfloat16),
                            preferred_element_type=jnp.float32)
                    + b1_ref[0, 0], 0.0)
    o = jnp.dot(h.astype(jnp.bfloat16), w2_ref[0].astype(jnp.bfloat16),
                preferred_element_type=jnp.float32) + b2_ref[0, 0]
    w = m_ref[0].T
    contrib = o * w

    @pl.when(e == 0)
    def _():
        out_ref[rows, :] = contrib

    @pl.when(e != 0)
    def _():
        out_ref[rows, :] += contrib


def kernel(features, receptivity, gate_W1, gate_b1, gate_W2, gate_b2,
           exp_W1, exp_b1, exp_W2, exp_b2):
    w1g = gate_W1.reshape(E, D_IN, D_HID)
    rec = receptivity[:, :, 0]

    m_t, ranks, tot = pl.pallas_call(
        _gate_kernel,
        grid=(E, NB),
        in_specs=[
            pl.BlockSpec((1, BLK, D_IN), lambda e, nb: (e, nb, 0)),
            pl.BlockSpec((1, D_IN, D_HID), lambda e, nb: (e, 0, 0)),
            pl.BlockSpec((1, D_HID), lambda e, nb: (0, 0)),
            pl.BlockSpec((D_HID, E), lambda e, nb: (0, 0)),
            pl.BlockSpec((1, E), lambda e, nb: (0, 0)),
            pl.BlockSpec((E, BLK), lambda e, nb: (0, nb)),
        ],
        out_specs=[
            pl.BlockSpec((E, BLK), lambda e, nb: (0, nb)),
            pl.BlockSpec((E, BLK), lambda e, nb: (0, nb)),
            pl.BlockSpec((1, E), lambda e, nb: (0, 0)),
        ],
        out_shape=[
            jax.ShapeDtypeStruct((E, N), jnp.float32),
            jax.ShapeDtypeStruct((E, N), jnp.int32),
            jax.ShapeDtypeStruct((1, E), jnp.int32),
        ],
        scratch_shapes=[pltpu.VMEM((N, D_HID), jnp.float32)],
    )(features, w1g, gate_b1.reshape(1, D_HID), gate_W2,
      gate_b2.reshape(1, E), rec)

    final_out = pl.pallas_call(
        _expert_kernel,
        grid=(E, ENB),
        in_specs=[
            pl.BlockSpec((1, EBLK, D_IN), lambda e, nb: (e, nb, 0)),
            pl.BlockSpec((1, D_IN, D_HID), lambda e, nb: (e, 0, 0)),
            pl.BlockSpec((1, 1, D_HID), lambda e, nb: (e, 0, 0)),
            pl.BlockSpec((1, D_HID, D_OUT), lambda e, nb: (e, 0, 0)),
            pl.BlockSpec((1, 1, D_OUT), lambda e, nb: (e, 0, 0)),
            pl.BlockSpec((1, 1, EBLK), lambda e, nb: (e, 0, nb)),
        ],
        out_specs=pl.BlockSpec((N, D_OUT), lambda e, nb: (0, 0)),
        out_shape=jax.ShapeDtypeStruct((N, D_OUT), jnp.float32),
    )(features, exp_W1, exp_b1.reshape(E, 1, D_HID), exp_W2,
      exp_b2.reshape(E, 1, D_OUT), m_t.reshape(E, 1, N))

    return final_out, ranks, tot.reshape(E)


# speedup vs baseline: 1.5823x; 1.0308x over previous
import jax
import jax.numpy as jnp
from jax.experimental import pallas as pl
from jax.experimental.pallas import tpu as pltpu

E = 8
TOP_K = 2
D_IN = 1024
D_HID = 512
D_OUT = 1024
N = 2048


def _fused_kernel(feat_ref, w1_ref, b1_ref, w2_ref, b2_ref, rec_ref,
                  ew1_ref, eb1_ref, ew2_ref, eb2_ref,
                  out_ref, ranks_ref, tot_ref, gh_scr, m_scr):
    s = pl.program_id(0)

    @pl.when(s < E)
    def _gate_phase():
        part = jnp.dot(feat_ref[0], w1_ref[0],
                       preferred_element_type=jnp.float32)

        @pl.when(s == 0)
        def _():
            gh_scr[...] = part

        @pl.when(s != 0)
        def _():
            gh_scr[...] += part

        @pl.when(s == E - 1)
        def _():
            gh = jnp.maximum(gh_scr[...] + b1_ref[0], 0.0)
            logits = jnp.dot(gh, w2_ref[...], preferred_element_type=jnp.float32)
            logits = logits + b2_ref[0]
            mx = jnp.max(logits, axis=1, keepdims=True)
            ex = jnp.exp(logits - mx)
            gw = ex / jnp.sum(ex, axis=1, keepdims=True)
            scores = gw + rec_ref[...].T

            col = jax.lax.broadcasted_iota(jnp.int32, (N, E), 1)
            v1 = jnp.max(scores, axis=1, keepdims=True)
            i1 = jnp.argmax(scores, axis=1).reshape(N, 1)
            masked = jnp.where(col == i1, -jnp.inf, scores)
            v2 = jnp.max(masked, axis=1, keepdims=True)
            i2 = jnp.argmax(masked, axis=1).reshape(N, 1)
            denom = v1 + v2
            m_scr[...] = (jnp.where(col == i1, v1 / denom, 0.0)
                          + jnp.where(col == i2, v2 / denom, 0.0))

            ranks = (2 - 2 * (col == i1).astype(jnp.int32)
                     - (col == i2).astype(jnp.int32))
            ranks_ref[...] = ranks.T
            tot_ref[...] = jnp.sum(ranks, axis=0, keepdims=True)

    @pl.when(s >= E)
    def _expert_phase():
        e = s - E
        x = feat_ref[0].astype(jnp.bfloat16)
        h = jnp.maximum(jnp.dot(x, ew1_ref[0].astype(jnp.bfloat16),
                                preferred_element_type=jnp.float32)
                        + eb1_ref[0, 0], 0.0)
        o = jnp.dot(h.astype(jnp.bfloat16), ew2_ref[0].astype(jnp.bfloat16),
                    preferred_element_type=jnp.float32) + eb2_ref[0, 0]
        col = jax.lax.broadcasted_iota(jnp.int32, (N, E), 1)
        w = jnp.sum(jnp.where(col == e, m_scr[...], 0.0), axis=1, keepdims=True)
        contrib = o * w

        @pl.when(s == E)
        def _():
            out_ref[...] = contrib

        @pl.when(s != E)
        def _():
            out_ref[...] += contrib


def kernel(features, receptivity, gate_W1, gate_b1, gate_W2, gate_b2,
           exp_W1, exp_b1, exp_W2, exp_b2):
    w1g = gate_W1.reshape(E, D_IN, D_HID)
    rec = receptivity[:, :, 0]

    final_out, ranks, tot = pl.pallas_call(
        _fused_kernel,
        grid=(2 * E,),
        in_specs=[
            pl.BlockSpec((1, N, D_IN), lambda s: (s % E, 0, 0)),
            pl.BlockSpec((1, D_IN, D_HID), lambda s: (jnp.minimum(s, E - 1), 0, 0)),
            pl.BlockSpec((1, D_HID), lambda s: (0, 0)),
            pl.BlockSpec((D_HID, E), lambda s: (0, 0)),
            pl.BlockSpec((1, E), lambda s: (0, 0)),
            pl.BlockSpec((E, N), lambda s: (0, 0)),
            pl.BlockSpec((1, D_IN, D_HID), lambda s: (jnp.maximum(s - E, 0), 0, 0)),
            pl.BlockSpec((1, 1, D_HID), lambda s: (jnp.maximum(s - E, 0), 0, 0)),
            pl.BlockSpec((1, D_HID, D_OUT), lambda s: (jnp.maximum(s - E, 0), 0, 0)),
            pl.BlockSpec((1, 1, D_OUT), lambda s: (jnp.maximum(s - E, 0), 0, 0)),
        ],
        out_specs=[
            pl.BlockSpec((N, D_OUT), lambda s: (0, 0)),
            pl.BlockSpec((E, N), lambda s: (0, 0)),
            pl.BlockSpec((1, E), lambda s: (0, 0)),
        ],
        out_shape=[
            jax.ShapeDtypeStruct((N, D_OUT), jnp.float32),
            jax.ShapeDtypeStruct((E, N), jnp.int32),
            jax.ShapeDtypeStruct((1, E), jnp.int32),
        ],
        scratch_shapes=[pltpu.VMEM((N, D_HID), jnp.float32),
                        pltpu.VMEM((N, E), jnp.float32)],
    )(features, w1g, gate_b1.reshape(1, D_HID), gate_W2,
      gate_b2.reshape(1, E), rec, exp_W1, exp_b1.reshape(E, 1, D_HID),
      exp_W2, exp_b2.reshape(E, 1, D_OUT))

    return final_out, ranks, tot.reshape(E)


# speedup vs baseline: 1.5869x; 1.0029x over previous
import jax
import jax.numpy as jnp
from jax.experimental import pallas as pl
from jax.experimental.pallas import tpu as pltpu

E = 8
TOP_K = 2
D_IN = 1024
D_HID = 512
D_OUT = 1024
N = 2048


def _fused_kernel(feat_ref, w1_ref, b1_ref, w2_ref, b2_ref, rec_ref,
                  ew1_ref, eb1_ref, ew2_ref, eb2_ref,
                  out_ref, ranks_ref, tot_ref, gh_scr, m_scr):
    s = pl.program_id(0)

    @pl.when(s < E)
    def _gate_phase():
        part = jnp.dot(feat_ref[0], w1_ref[0],
                       preferred_element_type=jnp.float32)

        @pl.when(s == 0)
        def _():
            gh_scr[...] = part

        @pl.when(s != 0)
        def _():
            gh_scr[...] += part

        @pl.when(s == E - 1)
        def _():
            gh = jnp.maximum(gh_scr[...] + b1_ref[0], 0.0)
            logits = jnp.dot(gh, w2_ref[...], preferred_element_type=jnp.float32)
            logits = logits + b2_ref[0]
            mx = jnp.max(logits, axis=1, keepdims=True)
            ex = jnp.exp(logits - mx)
            gw = ex / jnp.sum(ex, axis=1, keepdims=True)
            scores = gw + rec_ref[...].T

            col = jax.lax.broadcasted_iota(jnp.int32, (N, E), 1)
            v1 = jnp.max(scores, axis=1, keepdims=True)
            i1 = jnp.argmax(scores, axis=1).reshape(N, 1)
            masked = jnp.where(col == i1, -jnp.inf, scores)
            v2 = jnp.max(masked, axis=1, keepdims=True)
            i2 = jnp.argmax(masked, axis=1).reshape(N, 1)
            denom = v1 + v2
            m_scr[...] = (jnp.where(col == i1, v1 / denom, 0.0)
                          + jnp.where(col == i2, v2 / denom, 0.0))

            ranks = (2 - 2 * (col == i1).astype(jnp.int32)
                     - (col == i2).astype(jnp.int32))
            ranks_ref[...] = ranks.T
            tot_ref[...] = jnp.sum(ranks, axis=0, keepdims=True)

    @pl.when(s >= E)
    def _expert_phase():
        e = s - E
        x = feat_ref[0].astype(jnp.bfloat16)
        h = jnp.maximum(jnp.dot(x, ew1_ref[0].astype(jnp.bfloat16),
                                preferred_element_type=jnp.float32)
                        + eb1_ref[0, 0], 0.0).astype(jnp.bfloat16)
        o = jnp.dot(h, ew2_ref[0].astype(jnp.bfloat16),
                    preferred_element_type=jnp.float32) + eb2_ref[0, 0]
        col = jax.lax.broadcasted_iota(jnp.int32, (N, E), 1)
        w = jnp.sum(jnp.where(col == e, m_scr[...], 0.0), axis=1, keepdims=True)
        contrib = o * w

        @pl.when(s == E)
        def _():
            out_ref[...] = contrib

        @pl.when(s != E)
        def _():
            out_ref[...] += contrib


def kernel(features, receptivity, gate_W1, gate_b1, gate_W2, gate_b2,
           exp_W1, exp_b1, exp_W2, exp_b2):
    w1g = gate_W1.reshape(E, D_IN, D_HID)
    rec = receptivity[:, :, 0]

    final_out, ranks, tot = pl.pallas_call(
        _fused_kernel,
        grid=(2 * E,),
        in_specs=[
            pl.BlockSpec((1, N, D_IN), lambda s: (s % E, 0, 0)),
            pl.BlockSpec((1, D_IN, D_HID), lambda s: (jnp.minimum(s, E - 1), 0, 0)),
            pl.BlockSpec((1, D_HID), lambda s: (0, 0)),
            pl.BlockSpec((D_HID, E), lambda s: (0, 0)),
            pl.BlockSpec((1, E), lambda s: (0, 0)),
            pl.BlockSpec((E, N), lambda s: (0, 0)),
            pl.BlockSpec((1, D_IN, D_HID), lambda s: (jnp.maximum(s - E, 0), 0, 0)),
            pl.BlockSpec((1, 1, D_HID), lambda s: (jnp.maximum(s - E, 0), 0, 0)),
            pl.BlockSpec((1, D_HID, D_OUT), lambda s: (jnp.maximum(s - E, 0), 0, 0)),
            pl.BlockSpec((1, 1, D_OUT), lambda s: (jnp.maximum(s - E, 0), 0, 0)),
        ],
        out_specs=[
            pl.BlockSpec((N, D_OUT), lambda s: (0, 0)),
            pl.BlockSpec((E, N), lambda s: (0, 0)),
            pl.BlockSpec((1, E), lambda s: (0, 0)),
        ],
        out_shape=[
            jax.ShapeDtypeStruct((N, D_OUT), jnp.float32),
            jax.ShapeDtypeStruct((E, N), jnp.int32),
            jax.ShapeDtypeStruct((1, E), jnp.int32),
        ],
        scratch_shapes=[pltpu.VMEM((N, D_HID), jnp.float32),
                        pltpu.VMEM((N, E), jnp.float32)],
    )(features, w1g, gate_b1.reshape(1, D_HID), gate_W2,
      gate_b2.reshape(1, E), rec, exp_W1, exp_b1.reshape(E, 1, D_HID),
      exp_W2, exp_b2.reshape(E, 1, D_OUT))

    return final_out, ranks, tot.reshape(E)
